# Initial kernel scaffold; baseline (speedup 1.0000x reference)
import jax
import jax.numpy as jnp
from jax.experimental import pallas as pl


def kernel(entity_idx, adj_entity, adj_relation, entity_table, relation_table, att_w1, att_w2, att_w3, wx_w, wx_b, wc_w, wc_b):
    raise NotImplementedError("write your pallas kernel here")


# speedup vs baseline: 7.1953x; 7.1953x over previous
import functools

import jax
import jax.numpy as jnp
from jax import lax
from jax.experimental import pallas as pl
from jax.experimental.pallas import tpu as pltpu
from jax.experimental.pallas import tpu_sc as plsc

B, N_ENT, N_REL, D, K = 1024, 100000, 64, 128, 16
NC, NS, L = 2, 16, 16
NW = NC * NS
BPW = B // NW
NV = D // L
AROW = N_ENT // 8


def _leaky(x):
    return jnp.where(x >= 0, x, 0.2 * x)


def _renorm_scale(x):
    n = jnp.sqrt(jnp.sum(x * x, axis=1, keepdims=True))
    return jnp.where(n > 1.0, 1.0 / (n + 1e-7), 1.0)


def _bcast_lane(x, k):
    return jnp.take_along_axis(x, jnp.full((L,), k, jnp.int32), axis=0,
                               mode="promise_in_bounds")


def _lanesum(x):
    idx = lax.iota(jnp.int32, L)
    for sh in (1, 2, 4, 8):
        x = x + jnp.take_along_axis(x, jnp.bitwise_xor(idx, sh), axis=0,
                                    mode="promise_in_bounds")
    return x


def _lookup64(t4, idx):
    q = lax.shift_right_logical(idx, 4)
    m = jnp.bitwise_and(idx, 15)
    g = [jnp.take_along_axis(t, m, axis=0, mode="promise_in_bounds")
         for t in t4]
    return jnp.where(q == 0, g[0],
                     jnp.where(q == 1, g[1],
                               jnp.where(q == 2, g[2], g[3])))


def _dotT(a, b):
    return lax.dot_general(a, b, (((1,), (1,)), ((), ())),
                           preferred_element_type=jnp.float32)


def _scale_body(x_ref, o_ref):
    x = x_ref[...]
    o_ref[...] = x * _renorm_scale(x)


def _prescale(table):
    rows = table.shape[0]
    blk = 2000
    return pl.pallas_call(
        _scale_body,
        grid=(rows // blk,),
        in_specs=[pl.BlockSpec((blk, D), lambda i: (i, 0))],
        out_specs=pl.BlockSpec((blk, D), lambda i: (i, 0)),
        out_shape=jax.ShapeDtypeStruct((rows, D), jnp.float32),
    )(table)


def _sc_hs(entity_idx, adj_ent2d, st):
    mesh = plsc.VectorSubcoreMesh(core_axis_name="c", subcore_axis_name="s",
                                  num_cores=NC, num_subcores=NS)

    @functools.partial(
        pl.kernel,
        out_type=(jax.ShapeDtypeStruct((B, D), jnp.float32),
                  jax.ShapeDtypeStruct((B, D), jnp.float32)),
        mesh=mesh,
        scratch_types=[
            pltpu.VMEM((BPW,), jnp.int32),
            pltpu.VMEM((BPW,), jnp.int32),
            pltpu.VMEM((BPW, 8 * K), jnp.int32),
            pltpu.VMEM((BPW, K), jnp.int32),
            pltpu.VMEM((K, D), jnp.float32),
            pltpu.VMEM((BPW, D), jnp.float32),
            pltpu.VMEM((BPW, D), jnp.float32),
            pltpu.SemaphoreType.DMA,
            pltpu.SemaphoreType.DMA,
            pltpu.SemaphoreType.DMA,
        ],
    )
    def hs_kernel(eidx_hbm, adj_hbm, st_hbm, h_out, s1_out,
                  eidx_v, arow_v, abuf_v, e1_v, t1_v, h_v, s1_v,
                  sem0, sem1, sem2):
        wid = lax.axis_index("s") * NC + lax.axis_index("c")
        base = wid * BPW
        pltpu.sync_copy(eidx_hbm.at[pl.ds(base, BPW)], eidx_v)
        cp_h = pltpu.async_copy(st_hbm.at[eidx_v], h_v, sem0)
        for c in range(BPW // L):
            ch = eidx_v[pl.ds(c * L, L)]
            arow_v[pl.ds(c * L, L)] = lax.shift_right_logical(ch, 3)
        cp_a = pltpu.async_copy(adj_hbm.at[arow_v], abuf_v, sem1)
        cp_a.wait()
        for c in range(BPW // L):
            ch = eidx_v[pl.ds(c * L, L)]
            for i in range(L):
                b = c * L + i
                off = (ch[i] & 7) * K
                e1_v[b, :] = abuf_v[b, pl.ds(off, K)]

        def body(b, carry):
            pltpu.async_copy(st_hbm.at[e1_v[b]], t1_v, sem2).wait()
            for v in range(NV):
                acc = jnp.zeros((L,), jnp.float32)
                for k in range(K):
                    acc = acc + t1_v[k, pl.ds(v * L, L)]
                s1_v[b, pl.ds(v * L, L)] = acc
            return carry

        lax.fori_loop(0, BPW, body, 0)
        cp_h.wait()
        pltpu.sync_copy(h_v, h_out.at[pl.ds(base, BPW)])
        pltpu.sync_copy(s1_v, s1_out.at[pl.ds(base, BPW)])

    return hs_kernel(entity_idx, adj_ent2d, st)


def _att_body(h_ref, s1_ref, rel_ref, w1_ref, w2_ref, w3_ref, ew_ref):
    rel_raw = rel_ref[...]
    n = jnp.sqrt(jnp.sum(rel_raw * rel_raw, axis=1, keepdims=True))
    relr = rel_raw * jnp.where(n > 1.0, 1.0 / (n + 1e-7), 1.0)
    w1 = w1_ref[...]
    w2 = w2_ref[...]
    w3 = w3_ref[...]
    rB = _dotT(relr, w1[:, D:])

    def table(hv):
        hA = _dotT(hv, w1[:, :D])
        x = jax.nn.relu(hA[:, None, :] + rB[None, :, :])
        x = x.reshape(-1, D)
        y = jax.nn.relu(_dotT(x, w2))
        a = _dotT(y, w3)[:, 0]
        sig = 1.0 / (1.0 + jnp.exp(-a))
        return jnp.exp(sig).reshape(hv.shape[0], N_REL)

    ew_ref[...] = jnp.concatenate([table(h_ref[...]), table(s1_ref[...])],
                                  axis=1)


def _att_tables(h, s1, relation_table, att_w1, att_w2, att_w3):
    blk = 128
    full = lambda shape: pl.BlockSpec(shape, lambda i: (0, 0))
    return pl.pallas_call(
        _att_body,
        grid=(B // blk,),
        in_specs=[
            pl.BlockSpec((blk, D), lambda i: (i, 0)),
            pl.BlockSpec((blk, D), lambda i: (i, 0)),
            full((N_REL, D)),
            full((D, 2 * D)),
            full((D, D)),
            full((1, D)),
        ],
        out_specs=pl.BlockSpec((blk, 2 * N_REL), lambda i: (i, 0)),
        out_shape=jax.ShapeDtypeStruct((B, 2 * N_REL), jnp.float32),
    )(h, s1, relation_table, att_w1, att_w2, att_w3)


def _sc_agg(entity_idx, adj_ent2d, adj_rel2d, st, ew):
    mesh = plsc.VectorSubcoreMesh(core_axis_name="c", subcore_axis_name="s",
                                  num_cores=NC, num_subcores=NS)

    @functools.partial(
        pl.kernel,
        out_type=(jax.ShapeDtypeStruct((B, D), jnp.float32),
                  jax.ShapeDtypeStruct((B, D), jnp.float32)),
        mesh=mesh,
        scratch_types=[
            pltpu.VMEM((BPW,), jnp.int32),
            pltpu.VMEM((BPW,), jnp.int32),
            pltpu.VMEM((BPW, 8 * K), jnp.int32),
            pltpu.VMEM((BPW, 8 * K), jnp.int32),
            pltpu.VMEM((BPW, K), jnp.int32),
            pltpu.VMEM((BPW, K), jnp.int32),
            pltpu.VMEM((BPW * 2 * N_REL,), jnp.float32),
            pltpu.VMEM((K, 8 * K), jnp.int32),
            pltpu.VMEM((K, 8 * K), jnp.int32),
            pltpu.VMEM((K, K), jnp.int32),
            pltpu.VMEM((2, 8 * K), jnp.int32),
            pltpu.VMEM((K, D), jnp.float32),
            pltpu.VMEM((2, 8 * K, D), jnp.float32),
            pltpu.VMEM((BPW, D), jnp.float32),
            pltpu.VMEM((BPW, D), jnp.float32),
            pltpu.SemaphoreType.DMA,
            pltpu.SemaphoreType.DMA,
            pltpu.SemaphoreType.DMA,
            pltpu.SemaphoreType.DMA,
            pltpu.SemaphoreType.DMA,
        ],
    )
    def agg_kernel(eidx_hbm, adj_e_hbm, adj_r_hbm, st_hbm, ew_hbm,
                   v0_out, v1_out,
                   eidx_v, arow_v, abuf_v, rbuf_v, e1_v, r1_v, ew_v,
                   a2buf_v, r2buf_v, r2i_v, e2flat_v, t1_v, rows_v,
                   v0_v, v1_v,
                   semA, semB, semT, semR0, semR1):
        wid = lax.axis_index("s") * NC + lax.axis_index("c")
        base = wid * BPW
        pltpu.sync_copy(eidx_hbm.at[pl.ds(base, BPW)], eidx_v)
        pltpu.sync_copy(ew_hbm.at[pl.ds(base * 2 * N_REL, BPW * 2 * N_REL)],
                        ew_v)
        for c in range(BPW // L):
            ch = eidx_v[pl.ds(c * L, L)]
            arow_v[pl.ds(c * L, L)] = lax.shift_right_logical(ch, 3)
        pltpu.async_copy(adj_e_hbm.at[arow_v], abuf_v, semA)
        pltpu.async_copy(adj_r_hbm.at[arow_v], rbuf_v, semB)
        pltpu.make_async_copy(adj_e_hbm.at[arow_v], abuf_v, semA).wait()
        pltpu.make_async_copy(adj_r_hbm.at[arow_v], rbuf_v, semB).wait()
        for c in range(BPW // L):
            ch = eidx_v[pl.ds(c * L, L)]
            for i in range(L):
                b = c * L + i
                off = (ch[i] & 7) * K
                e1_v[b, :] = abuf_v[b, pl.ds(off, K)]
                r1_v[b, :] = rbuf_v[b, pl.ds(off, K)]

        def body(b, carry):
            e1row = e1_v[b]
            a2row = lax.shift_right_logical(e1row, 3)
            cpa = pltpu.async_copy(adj_e_hbm.at[a2row], a2buf_v, semA)
            cpb = pltpu.async_copy(adj_r_hbm.at[a2row], r2buf_v, semB)
            cpt = pltpu.async_copy(st_hbm.at[e1row], t1_v, semT)
            cpa.wait()
            for k in range(K):
                off = (e1row[k] & 7) * K
                e2flat_v[k // 8, pl.ds((k % 8) * K, K)] = a2buf_v[
                    k, pl.ds(off, K)]
            cr0 = pltpu.async_copy(st_hbm.at[e2flat_v.at[0]], rows_v.at[0],
                                   semR0)
            cr1 = pltpu.async_copy(st_hbm.at[e2flat_v.at[1]], rows_v.at[1],
                                   semR1)
            cpb.wait()
            for k in range(K):
                off = (e1row[k] & 7) * K
                r2i_v[k, :] = r2buf_v[k, pl.ds(off, K)]
            cpt.wait()
            ew0t = [ew_v[pl.ds(b * 2 * N_REL + q * L, L)] for q in range(4)]
            ew1t = [ew_v[pl.ds(b * 2 * N_REL + N_REL + q * L, L)]
                    for q in range(4)]

            w0 = _lookup64(ew0t, r1_v[b])
            acc = [jnp.zeros((L,), jnp.float32) for _ in range(NV)]
            for k in range(K):
                wk = _bcast_lane(w0, k)
                for v in range(NV):
                    acc[v] = acc[v] + wk * t1_v[k, pl.ds(v * L, L)]
            rden0 = 1.0 / _lanesum(w0)
            for v in range(NV):
                v0_v[b, pl.ds(v * L, L)] = acc[v] * rden0

            def make_gbody(half):
                def gbody(gl, carry2):
                    accs = list(carry2[:NV])
                    den = carry2[NV]
                    w = _lookup64(ew1t, r2i_v[half * 8 + gl])
                    den = den + w
                    for k in range(K):
                        wk = _bcast_lane(w, k)
                        for v in range(NV):
                            accs[v] = accs[v] + wk * rows_v[
                                half, gl * K + k, pl.ds(v * L, L)]
                    return (*accs, den)
                return gbody

            init = tuple(jnp.zeros((L,), jnp.float32)
                         for _ in range(NV + 1))
            cr0.wait()
            mid = lax.fori_loop(0, 8, make_gbody(0), init)
            cr1.wait()
            fin = lax.fori_loop(0, 8, make_gbody(1), mid)
            rden1 = 1.0 / _lanesum(fin[NV])
            for v in range(NV):
                v1_v[b, pl.ds(v * L, L)] = fin[v] * rden1
            return carry

        lax.fori_loop(0, BPW, body, 0)
        pltpu.sync_copy(v0_v, v0_out.at[pl.ds(base, BPW)])
        pltpu.sync_copy(v1_v, v1_out.at[pl.ds(base, BPW)])

    return agg_kernel(entity_idx, adj_ent2d, adj_rel2d, st, ew)


def _final_body(h_ref, s1_ref, v0p_ref, v1p_ref, wx_ref, wxb_ref,
                wc_ref, wcb_ref, o_ref):
    h = h_ref[...]
    s1 = s1_ref[...]
    wx = wx_ref[...]
    wxb = wxb_ref[...]
    wc = wc_ref[...]
    wcb = wcb_ref[...]
    v0 = _leaky(_dotT(v0p_ref[...], wx) + wxb)
    e1 = _leaky(_dotT(h, wc[:, :D]) + _dotT(v0, wc[:, D:]) + wcb)
    v1 = _leaky(_dotT(v1p_ref[...], wx) + wxb)
    e2 = _leaky(_dotT(s1, wc[:, :D]) + _dotT(v1, wc[:, D:]) + wcb)
    o_ref[...] = jnp.concatenate([e2, e1, h], axis=1)


def _final(h, s1, v0p, v1p, wx_w, wx_b, wc_w, wc_b):
    blk = 128
    full = lambda shape: pl.BlockSpec(shape, lambda i: tuple(0 for _ in shape))
    row = lambda: pl.BlockSpec((blk, D), lambda i: (i, 0))
    return pl.pallas_call(
        _final_body,
        grid=(B // blk,),
        in_specs=[
            row(), row(), row(), row(),
            full((D, D)),
            pl.BlockSpec((D,), lambda i: (0,)),
            full((D, 2 * D)),
            pl.BlockSpec((D,), lambda i: (0,)),
        ],
        out_specs=pl.BlockSpec((blk, 3 * D), lambda i: (i, 0)),
        out_shape=jax.ShapeDtypeStruct((B, 3 * D), jnp.float32),
    )(h, s1, v0p, v1p, wx_w, wx_b, wc_w, wc_b)


def kernel(entity_idx, adj_entity, adj_relation, entity_table, relation_table,
           att_w1, att_w2, att_w3, wx_w, wx_b, wc_w, wc_b):
    eidx = entity_idx.astype(jnp.int32)
    adj_e = adj_entity.astype(jnp.int32).reshape(AROW, 8 * K)
    adj_r = adj_relation.astype(jnp.int32).reshape(AROW, 8 * K)
    st = _prescale(entity_table)
    h, s1 = _sc_hs(eidx, adj_e, st)
    ew = _att_tables(h, s1, relation_table, att_w1, att_w2, att_w3)
    v0p, v1p = _sc_agg(eidx, adj_e, adj_r, st, ew.reshape(-1))
    return _final(h, s1, v0p, v1p, wx_w, wx_b, wc_w, wc_b)
